# baseline (speedup 1.0000x reference)
import functools

import jax
import jax.numpy as jnp
from jax import lax
from jax.experimental import pallas as pl
from jax.experimental.pallas import tpu as pltpu
from jax.experimental.pallas import tpu_sc as plsc

B, N = 8, 1024
P = N * N
E = B * P

NC, NS = 2, 16
NW = NC * NS
S = P // NW
VPB = N // 16


def _sc_body(out_hbm, row_v, col_v, sem):
    c = lax.axis_index("c")
    s = lax.axis_index("s")
    wid = s * NC + c
    lane = lax.iota(jnp.int32, 16)

    base = wid * (S // N)

    def fill_row(t, _):
        rval = jnp.broadcast_to(base + (t >> 3), (16,)).astype(jnp.int32)
        for u in range(8):
            row_v[pl.ds((t * 8 + u) * 16, 16)] = rval
        return 0

    lax.fori_loop(0, S // (16 * 8), fill_row, 0)

    copies = []
    for b in range(B):
        off = b * P + wid * S
        copies.append(pltpu.make_async_copy(row_v, out_hbm.at[0, pl.ds(off, S)], sem))
    for cp in copies:
        cp.start()

    def fill_col(t, _):
        for u in range(8):
            m = t * 8 + u
            col_v[pl.ds(m * 16, 16)] = lane + ((m * 16) & (N - 1))
        return 0

    lax.fori_loop(0, S // (16 * 8), fill_col, 0)

    for b in range(B):
        off = b * P + wid * S
        copies.append(pltpu.make_async_copy(col_v, out_hbm.at[1, pl.ds(off, S)], sem))
    for cp in copies[B:]:
        cp.start()
    for cp in copies:
        cp.wait()


_sc_edge_index = functools.partial(
    pl.kernel,
    out_type=jax.ShapeDtypeStruct((2, E), jnp.int32),
    mesh=plsc.VectorSubcoreMesh(core_axis_name="c", subcore_axis_name="s"),
    scratch_types=[
        pltpu.VMEM((S,), jnp.int32),
        pltpu.VMEM((S,), jnp.int32),
        pltpu.SemaphoreType.DMA,
    ],
)(_sc_body)


NBLK = 16
C = E // NBLK
BLK_R = C // N


def _tc_body(v_ref, attr_ref):
    attr_ref[...] = (0.5 * jnp.tanh(0.5 * v_ref[...]) + 0.5).reshape(C)


def kernel(V_Adap):
    edge_index = _sc_edge_index()
    v2d = V_Adap.reshape(B * N, N)
    edge_attr = pl.pallas_call(
        _tc_body,
        grid=(NBLK,),
        in_specs=[pl.BlockSpec((BLK_R, N), lambda i: (i, 0))],
        out_specs=pl.BlockSpec((C,), lambda i: (i,)),
        out_shape=jax.ShapeDtypeStruct((E,), jnp.float32),
    )(v2d)
    return edge_index, edge_attr


# speedup vs baseline: 1.0359x; 1.0359x over previous
import functools

import jax
import jax.numpy as jnp
from jax import lax
from jax.experimental import pallas as pl
from jax.experimental.pallas import tpu as pltpu
from jax.experimental.pallas import tpu_sc as plsc

B, N = 8, 1024
P = N * N
E = B * P

NC, NS = 2, 16
NW = NC * NS
S = P // NW
VPB = N // 16


def _sc_body(out_hbm, row_v, col_v, sem):
    c = lax.axis_index("c")
    s = lax.axis_index("s")
    wid = s * NC + c
    lane = lax.iota(jnp.int32, 16)

    base = wid * (S // N)

    def fill_row(t, _):
        rval = jnp.broadcast_to(base + (t >> 3), (16,)).astype(jnp.int32)
        for u in range(8):
            row_v[pl.ds((t * 8 + u) * 16, 16)] = rval
        return 0

    lax.fori_loop(0, S // (16 * 8), fill_row, 0)

    copies = []
    for b in range(B):
        off = b * P + wid * S
        copies.append(pltpu.make_async_copy(row_v, out_hbm.at[0, pl.ds(off, S)], sem))
    for cp in copies:
        cp.start()

    def fill_col(t, _):
        for u in range(8):
            m = t * 8 + u
            col_v[pl.ds(m * 16, 16)] = lane + ((m * 16) & (N - 1))
        return 0

    lax.fori_loop(0, S // (16 * 8), fill_col, 0)

    for b in range(B):
        off = b * P + wid * S
        copies.append(pltpu.make_async_copy(col_v, out_hbm.at[1, pl.ds(off, S)], sem))
    for cp in copies[B:]:
        cp.start()
    for cp in copies:
        cp.wait()


_sc_edge_index = functools.partial(
    pl.kernel,
    out_type=jax.ShapeDtypeStruct((2, E), jnp.int32),
    mesh=plsc.VectorSubcoreMesh(core_axis_name="c", subcore_axis_name="s"),
    scratch_types=[
        pltpu.VMEM((S,), jnp.int32),
        pltpu.VMEM((S,), jnp.int32),
        pltpu.SemaphoreType.DMA,
    ],
)(_sc_body)


NBLK = 8
C = E // NBLK
BLK_R = C // N


def _tc_body(v_ref, attr_ref):
    attr_ref[...] = (0.5 * jnp.tanh(0.5 * v_ref[...]) + 0.5).reshape(C)


def kernel(V_Adap):
    edge_index = _sc_edge_index()
    v2d = V_Adap.reshape(B * N, N)
    edge_attr = pl.pallas_call(
        _tc_body,
        grid=(NBLK,),
        in_specs=[pl.BlockSpec((BLK_R, N), lambda i: (i, 0))],
        out_specs=pl.BlockSpec((C,), lambda i: (i,)),
        out_shape=jax.ShapeDtypeStruct((E,), jnp.float32),
    )(v2d)
    return edge_index, edge_attr


# speedup vs baseline: 1.0473x; 1.0109x over previous
import functools

import jax
import jax.numpy as jnp
from jax import lax
from jax.experimental import pallas as pl
from jax.experimental.pallas import tpu as pltpu
from jax.experimental.pallas import tpu_sc as plsc

B, N = 8, 1024
P = N * N
E = B * P

NC, NS = 2, 16
NW = NC * NS
S = P // NW
VPB = N // 16


def _sc_body(out_hbm, row_v, col_v, sem):
    c = lax.axis_index("c")
    s = lax.axis_index("s")
    wid = s * NC + c
    lane = lax.iota(jnp.int32, 16)

    base = wid * (S // N)

    def fill_row(t, _):
        rval = jnp.broadcast_to(base + (t >> 3), (16,)).astype(jnp.int32)
        for u in range(8):
            row_v[pl.ds((t * 8 + u) * 16, 16)] = rval
        return 0

    lax.fori_loop(0, S // (16 * 8), fill_row, 0)

    copies = []
    for b in range(B):
        off = b * P + wid * S
        copies.append(pltpu.make_async_copy(row_v, out_hbm.at[0, pl.ds(off, S)], sem))
    for cp in copies:
        cp.start()

    def fill_col(t, _):
        for u in range(8):
            m = t * 8 + u
            col_v[pl.ds(m * 16, 16)] = lane + ((m * 16) & (N - 1))
        return 0

    lax.fori_loop(0, S // (16 * 8), fill_col, 0)

    for b in range(B):
        off = b * P + wid * S
        copies.append(pltpu.make_async_copy(col_v, out_hbm.at[1, pl.ds(off, S)], sem))
    for cp in copies[B:]:
        cp.start()
    for cp in copies:
        cp.wait()


_sc_edge_index = functools.partial(
    pl.kernel,
    out_type=jax.ShapeDtypeStruct((2, E), jnp.int32),
    mesh=plsc.VectorSubcoreMesh(core_axis_name="c", subcore_axis_name="s"),
    scratch_types=[
        pltpu.VMEM((S,), jnp.int32),
        pltpu.VMEM((S,), jnp.int32),
        pltpu.SemaphoreType.DMA,
    ],
)(_sc_body)


NBLK = 4
C = E // NBLK
BLK_R = C // N


def _tc_body(v_ref, attr_ref):
    attr_ref[...] = (0.5 * jnp.tanh(0.5 * v_ref[...]) + 0.5).reshape(C)


def kernel(V_Adap):
    edge_index = _sc_edge_index()
    v2d = V_Adap.reshape(B * N, N)
    edge_attr = pl.pallas_call(
        _tc_body,
        grid=(NBLK,),
        in_specs=[pl.BlockSpec((BLK_R, N), lambda i: (i, 0))],
        out_specs=pl.BlockSpec((C,), lambda i: (i,)),
        out_shape=jax.ShapeDtypeStruct((E,), jnp.float32),
    )(v2d)
    return edge_index, edge_attr


# speedup vs baseline: 1.0536x; 1.0061x over previous
import functools

import jax
import jax.numpy as jnp
from jax import lax
from jax.experimental import pallas as pl
from jax.experimental.pallas import tpu as pltpu
from jax.experimental.pallas import tpu_sc as plsc

B, N = 8, 1024
P = N * N
E = B * P

NC, NS = 2, 16
NW = NC * NS
S = P // NW
VPB = N // 16


def _sc_body(out_hbm, row_v, col_v, sem):
    c = lax.axis_index("c")
    s = lax.axis_index("s")
    wid = s * NC + c
    lane = lax.iota(jnp.int32, 16)

    base = wid * (S // N)
    Q = S // 4
    copies = []

    def fill_row(t, _):
        rval = jnp.broadcast_to(base + (t >> 3), (16,)).astype(jnp.int32)
        for u in range(8):
            row_v[pl.ds((t * 8 + u) * 16, 16)] = rval
        return 0

    def fill_col(t, _):
        for u in range(8):
            m = t * 8 + u
            col_v[pl.ds(m * 16, 16)] = lane + ((m * 16) & (N - 1))
        return 0

    TQ = S // (16 * 8) // 4
    for q in range(4):
        lax.fori_loop(q * TQ, (q + 1) * TQ, fill_row, 0)
        for b in range(B):
            off = b * P + wid * S + q * Q
            cp = pltpu.make_async_copy(
                row_v.at[pl.ds(q * Q, Q)], out_hbm.at[0, pl.ds(off, Q)], sem)
            cp.start()
            copies.append(cp)
    for q in range(4):
        lax.fori_loop(q * TQ, (q + 1) * TQ, fill_col, 0)
        for b in range(B):
            off = b * P + wid * S + q * Q
            cp = pltpu.make_async_copy(
                col_v.at[pl.ds(q * Q, Q)], out_hbm.at[1, pl.ds(off, Q)], sem)
            cp.start()
            copies.append(cp)
    for cp in copies:
        cp.wait()


_sc_edge_index = functools.partial(
    pl.kernel,
    out_type=jax.ShapeDtypeStruct((2, E), jnp.int32),
    mesh=plsc.VectorSubcoreMesh(core_axis_name="c", subcore_axis_name="s"),
    scratch_types=[
        pltpu.VMEM((S,), jnp.int32),
        pltpu.VMEM((S,), jnp.int32),
        pltpu.SemaphoreType.DMA,
    ],
)(_sc_body)


NBLK = 4
C = E // NBLK
BLK_R = C // N


def _tc_body(v_ref, attr_ref):
    attr_ref[...] = (0.5 * jnp.tanh(0.5 * v_ref[...]) + 0.5).reshape(C)


def kernel(V_Adap):
    edge_index = _sc_edge_index()
    v2d = V_Adap.reshape(B * N, N)
    edge_attr = pl.pallas_call(
        _tc_body,
        grid=(NBLK,),
        in_specs=[pl.BlockSpec((BLK_R, N), lambda i: (i, 0))],
        out_specs=pl.BlockSpec((C,), lambda i: (i,)),
        out_shape=jax.ShapeDtypeStruct((E,), jnp.float32),
    )(v2d)
    return edge_index, edge_attr


# speedup vs baseline: 1.0560x; 1.0023x over previous
import functools

import jax
import jax.numpy as jnp
from jax import lax
from jax.experimental import pallas as pl
from jax.experimental.pallas import tpu as pltpu
from jax.experimental.pallas import tpu_sc as plsc

B, N = 8, 1024
P = N * N
E = B * P

NC, NS = 2, 16
NW = NC * NS
S = P // NW
VPB = N // 16


def _sc_body(out_hbm, row_v, col_v, sem):
    c = lax.axis_index("c")
    s = lax.axis_index("s")
    wid = s * NC + c
    lane = lax.iota(jnp.int32, 16)

    base = wid * (S // N)
    Q = S // 4
    copies = []

    def fill_row(t, _):
        rval = jnp.broadcast_to(base + (t >> 3), (16,)).astype(jnp.int32)
        for u in range(8):
            row_v[pl.ds((t * 8 + u) * 16, 16)] = rval
        return 0

    def fill_col(t, _):
        for u in range(8):
            m = t * 8 + u
            col_v[pl.ds(m * 16, 16)] = lane + ((m * 16) & (N - 1))
        return 0

    TQ = S // (16 * 8) // 4

    def fire(plane, buf, q):
        def body(b, _):
            off = b * P + wid * S + q * Q
            pltpu.make_async_copy(
                buf.at[pl.ds(q * Q, Q)], out_hbm.at[plane, pl.ds(off, Q)], sem
            ).start()
            return 0

        lax.fori_loop(0, B, body, 0)

    for q in range(4):
        lax.fori_loop(q * TQ, (q + 1) * TQ, fill_row, 0)
        fire(0, row_v, q)
    for q in range(4):
        lax.fori_loop(q * TQ, (q + 1) * TQ, fill_col, 0)
        fire(1, col_v, q)

    def drain(t, _):
        pltpu.make_async_copy(
            row_v.at[pl.ds(0, Q)], out_hbm.at[0, pl.ds(0, Q)], sem
        ).wait()
        return 0

    lax.fori_loop(0, 8 * B, drain, 0)


_sc_edge_index = functools.partial(
    pl.kernel,
    out_type=jax.ShapeDtypeStruct((2, E), jnp.int32),
    mesh=plsc.VectorSubcoreMesh(core_axis_name="c", subcore_axis_name="s"),
    scratch_types=[
        pltpu.VMEM((S,), jnp.int32),
        pltpu.VMEM((S,), jnp.int32),
        pltpu.SemaphoreType.DMA,
    ],
)(_sc_body)


NBLK = 4
C = E // NBLK
BLK_R = C // N


def _tc_body(v_ref, attr_ref):
    attr_ref[...] = (0.5 * jnp.tanh(0.5 * v_ref[...]) + 0.5).reshape(C)


def kernel(V_Adap):
    edge_index = _sc_edge_index()
    v2d = V_Adap.reshape(B * N, N)
    edge_attr = pl.pallas_call(
        _tc_body,
        grid=(NBLK,),
        in_specs=[pl.BlockSpec((BLK_R, N), lambda i: (i, 0))],
        out_specs=pl.BlockSpec((C,), lambda i: (i,)),
        out_shape=jax.ShapeDtypeStruct((E,), jnp.float32),
    )(v2d)
    return edge_index, edge_attr
